# Initial kernel scaffold; baseline (speedup 1.0000x reference)
import jax
import jax.numpy as jnp
from jax.experimental import pallas as pl


def kernel(feature1, feature2, W_gat, att_src, att_dst, b_gat, W_fc, b_fc):
    raise NotImplementedError("write your pallas kernel here")


# speedup vs baseline: 2.0201x; 2.0201x over previous
import functools

import jax
import jax.numpy as jnp
from jax.experimental import pallas as pl

B_TILE = 4000


def _gat_kernel(f1_ref, f2_ref, wg_ref, asrc_ref, adst_ref, bgat_ref,
                wf_ref, bfc_ref, out_ref):
    f1 = f1_ref[...]
    f2 = f2_ref[...]
    wg = wg_ref[...]
    dn = (((1,), (1,)), ((), ()))
    h0 = jax.lax.dot_general(f1, wg, dn, preferred_element_type=jnp.float32)
    h1 = jax.lax.dot_general(f2, wg, dn, preferred_element_type=jnp.float32)

    asrc = asrc_ref[...]
    adst = adst_ref[...]
    s0 = jnp.sum(h0 * asrc, axis=-1, keepdims=True)
    s1 = jnp.sum(h1 * asrc, axis=-1, keepdims=True)
    d0 = jnp.sum(h0 * adst, axis=-1, keepdims=True)
    d1 = jnp.sum(h1 * adst, axis=-1, keepdims=True)

    def lrelu(x):
        return jnp.where(x > 0, x, 0.2 * x)

    e10 = lrelu(s1 + d0)
    e00 = lrelu(s0 + d0)
    m0 = jnp.maximum(e10, e00)
    w10 = jnp.exp(e10 - m0)
    w00 = jnp.exp(e00 - m0)
    den0 = w10 + w00
    e01 = lrelu(s0 + d1)
    e11 = lrelu(s1 + d1)
    m1 = jnp.maximum(e01, e11)
    w01 = jnp.exp(e01 - m1)
    w11 = jnp.exp(e11 - m1)
    den1 = w01 + w11

    c0 = w00 / den0 + w01 / den1
    c1 = w10 / den0 + w11 / den1

    m = 0.5 * (c0 * h0 + c1 * h1) + bgat_ref[...]
    wf = wf_ref[...]
    res = jax.lax.dot_general(m, wf, dn, preferred_element_type=jnp.float32)
    out_ref[...] = res + bfc_ref[...]


@jax.jit
def kernel(feature1, feature2, W_gat, att_src, att_dst, b_gat, W_fc, b_fc):
    b, f_dim = feature1.shape
    h_dim = W_gat.shape[0]
    n_tiles = b // B_TILE

    row_spec = pl.BlockSpec((B_TILE, f_dim), lambda i: (i, 0))
    full = lambda shape: pl.BlockSpec(shape, lambda i: (0, 0))

    return pl.pallas_call(
        _gat_kernel,
        grid=(n_tiles,),
        in_specs=[
            row_spec,
            row_spec,
            full((h_dim, f_dim)),
            full((1, h_dim)),
            full((1, h_dim)),
            full((1, h_dim)),
            full((f_dim, h_dim)),
            full((1, f_dim)),
        ],
        out_specs=row_spec,
        out_shape=jax.ShapeDtypeStruct((b, f_dim), jnp.float32),
    )(feature1, feature2, W_gat,
      att_src.reshape(1, h_dim), att_dst.reshape(1, h_dim),
      b_gat.reshape(1, h_dim), W_fc, b_fc.reshape(1, f_dim))
